# baseline (speedup 1.0000x reference)
import jax
import jax.numpy as jnp
from jax import lax
from jax.experimental import pallas as pl
from jax.experimental.pallas import tpu as pltpu

_VMEM_LIMIT = 100 * 1024 * 1024
_EPS = 1e-6


def _scale_shift(s, ss, gm_ref, g_ref, bt_ref, inv):
    mean = jnp.dot(s.astype(jnp.bfloat16), gm_ref[...],
                   preferred_element_type=jnp.float32) * inv
    ex2 = jnp.dot(ss.astype(jnp.bfloat16), gm_ref[...],
                  preferred_element_type=jnp.float32) * inv
    var = ex2 - mean * mean
    scale = g_ref[...] * lax.rsqrt(var + _EPS)
    shift = bt_ref[...] - mean * scale
    return scale, shift


def _conv_chunks(load, scale, shift, w_ref, cb_ref, S, cin, rt):
    for r0 in range(0, S, rt):
        lo = max(r0 - 1, 0)
        hi = min(r0 + rt + 1, S)
        y = load(lo * S, hi * S) * scale + shift
        y = 0.5 * y * (1.0 + jnp.tanh(0.5 * y))
        yb = y.astype(jnp.bfloat16).reshape(hi - lo, S, cin)
        sl = jnp.pad(yb, ((1 - (r0 - lo), 1 - (hi - r0 - rt)),
                          (1, 1), (0, 0)))
        patches = jnp.concatenate(
            [sl[dy:dy + rt, dx:dx + S, :].reshape(rt * S, cin)
             for dy in range(3) for dx in range(3)], axis=-1)
        yield r0, (jnp.dot(patches, w_ref[...],
                           preferred_element_type=jnp.float32) + cb_ref[...])


def _make_net_kernel(S0, cin0, meta):
    def body(*refs):
        xu_ref, wst_ref, bst_ref, gm_a, gm_b, gm_c = refs[:6]
        gms = {}
        for r in (gm_a, gm_b, gm_c):
            gms[r.shape[0]] = r
        k = 6
        bparams = []
        for (S, cin, cout, cg1, cg2, has_proj, do_pool) in meta:
            nper = 8 + (2 if has_proj else 0)
            bparams.append(refs[k:k + nper])
            k += nper
        f_refs = refs[k:k + 3]
        x0_s, h0_s, p0_s, h1_s, p1_s, h2_s = refs[k + 3:k + 9]
        h_scr = [h0_s, h1_s, h2_s]
        in_scr = [x0_s, p0_s, p1_s]

        acc4 = jnp.dot(xu_ref[0], wst_ref[...],
                       preferred_element_type=jnp.float32) + bst_ref[...]
        x0_s[...] = acc4.reshape(S0 * S0, cin0).astype(x0_s.dtype)
        s4 = jnp.sum(acc4, axis=0, keepdims=True)
        ss4 = jnp.sum(acc4 * acc4, axis=0, keepdims=True)
        s = sum(s4[:, p * cin0:(p + 1) * cin0] for p in range(4))
        ss = sum(ss4[:, p * cin0:(p + 1) * cin0] for p in range(4))

        for i, (S, cin, cout, cg1, cg2, has_proj, do_pool) in enumerate(meta):
            prm = bparams[i]
            if has_proj:
                (g1, b1, w1, cb1, g2, b2, w2, cb2, scw, scb) = prm
            else:
                (g1, b1, w1, cb1, g2, b2, w2, cb2) = prm
                scw = scb = None
            x_s = in_scr[i]
            h_s = h_scr[i]
            rt = max(2, min(16, 1024 // S, S))
            hw = S * S

            scale, shift = _scale_shift(s, ss, gms[cin], g1, b1,
                                        1.0 / float(hw * cg1))
            ts = tss = 0.0
            for r0, a1 in _conv_chunks(
                    lambda a, b: x_s[a:b, :].astype(jnp.float32),
                    scale, shift, w1, cb1, S, cin, rt):
                h_s[r0 * S:(r0 + rt) * S, :] = a1.astype(h_s.dtype)
                ts = ts + jnp.sum(a1, axis=0, keepdims=True)
                tss = tss + jnp.sum(a1 * a1, axis=0, keepdims=True)

            scale, shift = _scale_shift(ts, tss, gms[cout], g2, b2,
                                        1.0 / float(hw * cg2))
            res = None
            if has_proj:
                res = jnp.dot(x_s[...], scw[...],
                              preferred_element_type=jnp.float32) + scb[...]
            ps = pss = 0.0
            for r0, a2 in _conv_chunks(
                    lambda a, b: h_s[a:b, :].astype(jnp.float32),
                    scale, shift, w2, cb2, S, cout, rt):
                a, b = r0 * S, (r0 + rt) * S
                if has_proj:
                    a2 = a2 + res[a:b, :]
                else:
                    a2 = a2 + x_s[a:b, :].astype(jnp.float32)
                f_refs[i][0, a:b, :] = a2.astype(f_refs[i].dtype)
                if do_pool:
                    v = a2.reshape(rt // 2, 2, S // 2, 2, cout)
                    pq = 0.25 * (v[:, 0, :, 0, :] + v[:, 0, :, 1, :]
                                 + v[:, 1, :, 0, :] + v[:, 1, :, 1, :])
                    pf = pq.reshape(rt * S // 4, cout)
                    in_scr[i + 1][(r0 // 2) * (S // 2):
                                  (r0 // 2 + rt // 2) * (S // 2), :] = (
                        pf.astype(in_scr[i + 1].dtype))
                    ps = ps + jnp.sum(pf, axis=0, keepdims=True)
                    pss = pss + jnp.sum(pf * pf, axis=0, keepdims=True)
            s, ss = ps, pss

    return body


def kernel(x, conv_in_w, conv_in_b,
           r0_gn1_gamma, r0_gn1_beta, r0_conv1_w, r0_conv1_b,
           r0_gn2_gamma, r0_gn2_beta, r0_conv2_w, r0_conv2_b,
           r1_gn1_gamma, r1_gn1_beta, r1_conv1_w, r1_conv1_b,
           r1_gn2_gamma, r1_gn2_beta, r1_conv2_w, r1_conv2_b,
           r1_sc_w, r1_sc_b,
           r2_gn1_gamma, r2_gn1_beta, r2_conv1_w, r2_conv1_b,
           r2_gn2_gamma, r2_gn2_beta, r2_conv2_w, r2_conv2_b,
           r2_sc_w, r2_sc_b):
    groups = 32
    f32, bf16 = jnp.float32, jnp.bfloat16
    B, c0, hr, wr = x.shape
    H, W = hr // 2, wr // 2
    cu = c0 * 4
    xu = (x.reshape(B, c0, H, 2, W, 2).transpose(0, 2, 4, 1, 3, 5)
          .reshape(B, H * W, cu).astype(bf16))

    raw = [
        dict(gn1=(r0_gn1_gamma, r0_gn1_beta), w1=r0_conv1_w, b1=r0_conv1_b,
             gn2=(r0_gn2_gamma, r0_gn2_beta), w2=r0_conv2_w, b2=r0_conv2_b,
             sc=None),
        dict(gn1=(r1_gn1_gamma, r1_gn1_beta), w1=r1_conv1_w, b1=r1_conv1_b,
             gn2=(r1_gn2_gamma, r1_gn2_beta), w2=r1_conv2_w, b2=r1_conv2_b,
             sc=(r1_sc_w, r1_sc_b)),
        dict(gn1=(r2_gn1_gamma, r2_gn1_beta), w1=r2_conv1_w, b1=r2_conv1_b,
             gn2=(r2_gn2_gamma, r2_gn2_beta), w2=r2_conv2_w, b2=r2_conv2_b,
             sc=(r2_sc_w, r2_sc_b)),
    ]

    cin0 = conv_in_w.shape[1]
    meta = []
    args = []
    in_specs = []

    def _add(arr):
        shp = arr.shape
        in_specs.append(pl.BlockSpec(shp, lambda *_: (0,) * len(shp)))
        args.append(arr)

    S = H
    for bp in raw:
        cin, cout = bp["w1"].shape[2], bp["w1"].shape[3]
        meta.append((S, cin, cout, cin // groups, cout // groups,
                     bp["sc"] is not None, bp is not raw[-1]))
        S //= 2

    in_specs.append(pl.BlockSpec((1, H * W // 4, 4 * cu), lambda b: (b, 0, 0)))
    args.append(xu.reshape(B, H * W // 4, 4 * cu))
    _add(jnp.kron(jnp.eye(4, dtype=f32), conv_in_w).astype(bf16))
    _add(jnp.tile(conv_in_b.reshape(1, cin0), (1, 4)).astype(f32))
    for c in sorted({m[1] for m in meta} | {m[2] for m in meta}):
        gidx = jnp.arange(c) // (c // groups)
        _add((gidx[:, None] == gidx[None, :]).astype(bf16))
    for bp, (S, cin, cout, *_r) in zip(raw, meta):
        _add(bp["gn1"][0].reshape(1, cin).astype(f32))
        _add(bp["gn1"][1].reshape(1, cin).astype(f32))
        _add(bp["w1"].reshape(9 * cin, cout).astype(bf16))
        _add(bp["b1"].reshape(1, cout).astype(f32))
        _add(bp["gn2"][0].reshape(1, cout).astype(f32))
        _add(bp["gn2"][1].reshape(1, cout).astype(f32))
        _add(bp["w2"].reshape(9 * cout, cout).astype(bf16))
        _add(bp["b2"].reshape(1, cout).astype(f32))
        if bp["sc"] is not None:
            _add(bp["sc"][0].astype(bf16))
            _add(bp["sc"][1].reshape(1, cout).astype(f32))

    out_shapes = []
    out_specs = []
    for (S, cin, cout, *_r) in meta:
        out_shapes.append(jax.ShapeDtypeStruct((B, S * S, cout), f32))
        out_specs.append(pl.BlockSpec((1, S * S, cout), lambda b: (b, 0, 0)))

    (S0, ci0, co0, *_), (S1, ci1, co1, *_), (S2, ci2, co2, *_) = meta
    scratch = [pltpu.VMEM((S0 * S0, ci0), bf16),
               pltpu.VMEM((S0 * S0, co0), bf16),
               pltpu.VMEM((S1 * S1, ci1), bf16),
               pltpu.VMEM((S1 * S1, co1), bf16),
               pltpu.VMEM((S2 * S2, ci2), bf16),
               pltpu.VMEM((S2 * S2, co2), bf16)]

    feats = pl.pallas_call(
        _make_net_kernel(H, cin0, meta),
        out_shape=tuple(out_shapes),
        grid=(B,),
        in_specs=in_specs,
        out_specs=tuple(out_specs),
        scratch_shapes=scratch,
        compiler_params=pltpu.CompilerParams(
            dimension_semantics=("parallel",),
            vmem_limit_bytes=_VMEM_LIMIT),
    )(*args)

    return [f.reshape(B, Si * Si, co).reshape(B, Si, Si, co)
            .transpose(0, 3, 1, 2)
            for f, (Si, ci, co, *_r) in zip(feats, meta)]


# speedup vs baseline: 1.1718x; 1.1718x over previous
import jax
import jax.numpy as jnp
from jax import lax
from jax.experimental import pallas as pl
from jax.experimental.pallas import tpu as pltpu

_VMEM_LIMIT = 100 * 1024 * 1024
_EPS = 1e-6


def _scale_shift(s, ss, gm_ref, g_ref, bt_ref, inv):
    mean = jnp.dot(s.astype(jnp.bfloat16), gm_ref[...],
                   preferred_element_type=jnp.float32) * inv
    ex2 = jnp.dot(ss.astype(jnp.bfloat16), gm_ref[...],
                  preferred_element_type=jnp.float32) * inv
    var = ex2 - mean * mean
    scale = g_ref[...] * lax.rsqrt(var + _EPS)
    shift = bt_ref[...] - mean * scale
    return scale, shift


def _conv_chunks(load, scale, shift, w_ref, cb_ref, S, cin, rt):
    for r0 in range(0, S, rt):
        lo = max(r0 - 1, 0)
        hi = min(r0 + rt + 1, S)
        y = load(lo * S, hi * S) * scale + shift
        y = 0.5 * y * (1.0 + jnp.tanh(0.5 * y))
        yb = y.astype(jnp.bfloat16).reshape(hi - lo, S, cin)
        sl = jnp.pad(yb, ((1 - (r0 - lo), 1 - (hi - r0 - rt)),
                          (1, 1), (0, 0)))
        patches = jnp.concatenate(
            [sl[dy:dy + rt, dx:dx + S, :].reshape(rt * S, cin)
             for dy in range(3) for dx in range(3)], axis=-1)
        yield r0, (jnp.dot(patches, w_ref[...],
                           preferred_element_type=jnp.float32) + cb_ref[...])


def _make_net_kernel(S0, cin0, meta):
    def body(*refs):
        xu_ref, wst_ref, bst_ref, gm_a, gm_b, gm_c = refs[:6]
        gms = {}
        for r in (gm_a, gm_b, gm_c):
            gms[r.shape[0]] = r
        k = 6
        bparams = []
        for (S, cin, cout, cg1, cg2, has_proj, do_pool) in meta:
            nper = 8 + (2 if has_proj else 0)
            bparams.append(refs[k:k + nper])
            k += nper
        f_refs = refs[k:k + 3]
        x0_s, h0_s, p0_s, h1_s, p1_s, h2_s = refs[k + 3:k + 9]
        h_scr = [h0_s, h1_s, h2_s]
        in_scr = [x0_s, p0_s, p1_s]

        acc4 = jnp.dot(xu_ref[0], wst_ref[...],
                       preferred_element_type=jnp.float32) + bst_ref[...]
        x0_s[...] = acc4.reshape(S0 * S0, cin0).astype(x0_s.dtype)
        s4 = jnp.sum(acc4, axis=0, keepdims=True)
        ss4 = jnp.sum(acc4 * acc4, axis=0, keepdims=True)
        s = sum(s4[:, p * cin0:(p + 1) * cin0] for p in range(4))
        ss = sum(ss4[:, p * cin0:(p + 1) * cin0] for p in range(4))

        for i, (S, cin, cout, cg1, cg2, has_proj, do_pool) in enumerate(meta):
            prm = bparams[i]
            if has_proj:
                (g1, b1, w1, cb1, g2, b2, w2, cb2, scw, scb) = prm
            else:
                (g1, b1, w1, cb1, g2, b2, w2, cb2) = prm
                scw = scb = None
            x_s = in_scr[i]
            h_s = h_scr[i]
            rt = max(2, min(16, 1024 // S, S))
            hw = S * S

            scale, shift = _scale_shift(s, ss, gms[cin], g1, b1,
                                        1.0 / float(hw * cg1))
            ts = tss = 0.0
            for r0, a1 in _conv_chunks(
                    lambda a, b: x_s[a:b, :].astype(jnp.float32),
                    scale, shift, w1, cb1, S, cin, rt):
                h_s[r0 * S:(r0 + rt) * S, :] = a1.astype(h_s.dtype)
                ts = ts + jnp.sum(a1, axis=0, keepdims=True)
                tss = tss + jnp.sum(a1 * a1, axis=0, keepdims=True)

            scale, shift = _scale_shift(ts, tss, gms[cout], g2, b2,
                                        1.0 / float(hw * cg2))
            ps = pss = 0.0
            for r0, a2 in _conv_chunks(
                    lambda a, b: h_s[a:b, :].astype(jnp.float32),
                    scale, shift, w2, cb2, S, cout, rt):
                a, b = r0 * S, (r0 + rt) * S
                if has_proj:
                    a2 = a2 + (jnp.dot(x_s[a:b, :], scw[...],
                                       preferred_element_type=jnp.float32)
                               + scb[...])
                else:
                    a2 = a2 + x_s[a:b, :].astype(jnp.float32)
                f_refs[i][0, a:b, :] = a2.astype(f_refs[i].dtype)
                if do_pool:
                    v = a2.reshape(rt // 2, 2, S // 2, 2, cout)
                    pq = 0.25 * (v[:, 0, :, 0, :] + v[:, 0, :, 1, :]
                                 + v[:, 1, :, 0, :] + v[:, 1, :, 1, :])
                    pf = pq.reshape(rt * S // 4, cout)
                    in_scr[i + 1][(r0 // 2) * (S // 2):
                                  (r0 // 2 + rt // 2) * (S // 2), :] = (
                        pf.astype(in_scr[i + 1].dtype))
                    ps = ps + jnp.sum(pf, axis=0, keepdims=True)
                    pss = pss + jnp.sum(pf * pf, axis=0, keepdims=True)
            s, ss = ps, pss

    return body


def kernel(x, conv_in_w, conv_in_b,
           r0_gn1_gamma, r0_gn1_beta, r0_conv1_w, r0_conv1_b,
           r0_gn2_gamma, r0_gn2_beta, r0_conv2_w, r0_conv2_b,
           r1_gn1_gamma, r1_gn1_beta, r1_conv1_w, r1_conv1_b,
           r1_gn2_gamma, r1_gn2_beta, r1_conv2_w, r1_conv2_b,
           r1_sc_w, r1_sc_b,
           r2_gn1_gamma, r2_gn1_beta, r2_conv1_w, r2_conv1_b,
           r2_gn2_gamma, r2_gn2_beta, r2_conv2_w, r2_conv2_b,
           r2_sc_w, r2_sc_b):
    groups = 32
    f32, bf16 = jnp.float32, jnp.bfloat16
    B, c0, hr, wr = x.shape
    H, W = hr // 2, wr // 2
    cu = c0 * 4
    xu = (x.reshape(B, c0, H, 2, W, 2).transpose(0, 2, 4, 1, 3, 5)
          .reshape(B, H * W, cu).astype(bf16))

    raw = [
        dict(gn1=(r0_gn1_gamma, r0_gn1_beta), w1=r0_conv1_w, b1=r0_conv1_b,
             gn2=(r0_gn2_gamma, r0_gn2_beta), w2=r0_conv2_w, b2=r0_conv2_b,
             sc=None),
        dict(gn1=(r1_gn1_gamma, r1_gn1_beta), w1=r1_conv1_w, b1=r1_conv1_b,
             gn2=(r1_gn2_gamma, r1_gn2_beta), w2=r1_conv2_w, b2=r1_conv2_b,
             sc=(r1_sc_w, r1_sc_b)),
        dict(gn1=(r2_gn1_gamma, r2_gn1_beta), w1=r2_conv1_w, b1=r2_conv1_b,
             gn2=(r2_gn2_gamma, r2_gn2_beta), w2=r2_conv2_w, b2=r2_conv2_b,
             sc=(r2_sc_w, r2_sc_b)),
    ]

    cin0 = conv_in_w.shape[1]
    meta = []
    args = []
    in_specs = []

    def _add(arr):
        shp = arr.shape
        in_specs.append(pl.BlockSpec(shp, lambda *_: (0,) * len(shp)))
        args.append(arr)

    S = H
    for bp in raw:
        cin, cout = bp["w1"].shape[2], bp["w1"].shape[3]
        meta.append((S, cin, cout, cin // groups, cout // groups,
                     bp["sc"] is not None, bp is not raw[-1]))
        S //= 2

    in_specs.append(pl.BlockSpec((1, H * W // 4, 4 * cu), lambda b: (b, 0, 0)))
    args.append(xu.reshape(B, H * W // 4, 4 * cu))
    _add(jnp.kron(jnp.eye(4, dtype=f32), conv_in_w).astype(bf16))
    _add(jnp.tile(conv_in_b.reshape(1, cin0), (1, 4)).astype(f32))
    for c in sorted({m[1] for m in meta} | {m[2] for m in meta}):
        gidx = jnp.arange(c) // (c // groups)
        _add((gidx[:, None] == gidx[None, :]).astype(bf16))
    for bp, (S, cin, cout, *_r) in zip(raw, meta):
        _add(bp["gn1"][0].reshape(1, cin).astype(f32))
        _add(bp["gn1"][1].reshape(1, cin).astype(f32))
        _add(bp["w1"].reshape(9 * cin, cout).astype(bf16))
        _add(bp["b1"].reshape(1, cout).astype(f32))
        _add(bp["gn2"][0].reshape(1, cout).astype(f32))
        _add(bp["gn2"][1].reshape(1, cout).astype(f32))
        _add(bp["w2"].reshape(9 * cout, cout).astype(bf16))
        _add(bp["b2"].reshape(1, cout).astype(f32))
        if bp["sc"] is not None:
            _add(bp["sc"][0].astype(bf16))
            _add(bp["sc"][1].reshape(1, cout).astype(f32))

    out_shapes = []
    out_specs = []
    for (S, cin, cout, *_r) in meta:
        out_shapes.append(jax.ShapeDtypeStruct((B, S * S, cout), f32))
        out_specs.append(pl.BlockSpec((1, S * S, cout), lambda b: (b, 0, 0)))

    (S0, ci0, co0, *_), (S1, ci1, co1, *_), (S2, ci2, co2, *_) = meta
    scratch = [pltpu.VMEM((S0 * S0, ci0), bf16),
               pltpu.VMEM((S0 * S0, co0), bf16),
               pltpu.VMEM((S1 * S1, ci1), bf16),
               pltpu.VMEM((S1 * S1, co1), bf16),
               pltpu.VMEM((S2 * S2, ci2), bf16),
               pltpu.VMEM((S2 * S2, co2), bf16)]

    feats = pl.pallas_call(
        _make_net_kernel(H, cin0, meta),
        out_shape=tuple(out_shapes),
        grid=(B,),
        in_specs=in_specs,
        out_specs=tuple(out_specs),
        scratch_shapes=scratch,
        compiler_params=pltpu.CompilerParams(
            dimension_semantics=("parallel",),
            vmem_limit_bytes=_VMEM_LIMIT),
    )(*args)

    return [f.reshape(B, Si * Si, co).reshape(B, Si, Si, co)
            .transpose(0, 3, 1, 2)
            for f, (Si, ci, co, *_r) in zip(feats, meta)]
